# baseline (speedup 1.0000x reference)
import jax
import jax.numpy as jnp
from jax import lax
from jax.experimental import pallas as pl
from jax.experimental.pallas import tpu as pltpu
from jax.experimental.pallas import tpu_sc as plsc

NUM_CODES = 1000000
DIM = 16
PAD_IDX = 0
BATCH = 16384
HIST = 200

NC = 2
NS = 16
NW = NC * NS
ROWS_PER_WORKER = BATCH // NW
CHUNK = 16
NCHUNK = ROWS_PER_WORKER // CHUNK

TBLK = 1024
TSTEPS = 30
NBLK_FULL = TSTEPS * NW
MAIN_CODES = NBLK_FULL * TBLK
TAIL = NUM_CODES - MAIN_CODES
OROWS = TBLK * DIM // 128
ROWS128 = NUM_CODES * DIM // 128
TAILROWS = TAIL * DIM // 128


def _transpose_body(tT_hbm, tail_hbm, out_hbm,
                    tba, tbb, oba, obb, sia, sib, soa, sob):
    wid = lax.axis_index("s") * NC + lax.axis_index("c")
    lanes = lax.iota(jnp.int32, 16)

    def col0(s):
        return (s * NW + wid) * TBLK

    def orow0(s):
        return (s * NW + wid) * OROWS

    def fire_in(s, tb, si):
        pltpu.async_copy(tT_hbm.at[:, pl.ds(col0(s), TBLK)],
                         tb.at[:, pl.ds(0, TBLK)], si)

    def wait_in(tb, si):
        pltpu.make_async_copy(tT_hbm.at[:, pl.ds(0, TBLK)],
                              tb.at[:, pl.ds(0, TBLK)], si).wait()

    def fire_out(s, ob, so):
        pltpu.async_copy(ob, out_hbm.at[pl.ds(orow0(s), OROWS)], so)

    def wait_out(ob, so):
        pltpu.make_async_copy(ob, out_hbm.at[pl.ds(0, OROWS)], so).wait()

    def do_block(tb, ob):
        def kb(j, colv):
            cv = colv
            for kk in range(8):
                e = plsc.load_gather(tb, [lanes, cv])
                ob[j, pl.ds(kk * DIM, DIM)] = e
                cv = cv + 1
            return cv
        lax.fori_loop(0, OROWS, kb, jnp.zeros((16,), jnp.int32))

    fire_in(0, tba, sia)

    def body(i, carry):
        s0 = 2 * i
        s1 = 2 * i + 1
        fire_in(s1, tbb, sib)
        wait_in(tba, sia)

        @pl.when(i > 0)
        def _():
            wait_out(oba, soa)

        do_block(tba, oba)
        fire_out(s0, oba, soa)

        @pl.when(s0 + 2 <= TSTEPS - 1)
        def _():
            fire_in(s0 + 2, tba, sia)

        wait_in(tbb, sib)

        @pl.when(i > 0)
        def _():
            wait_out(obb, sob)

        do_block(tbb, obb)
        fire_out(s1, obb, sob)
        return carry

    lax.fori_loop(0, TSTEPS // 2, body, 0)

    wait_out(oba, soa)
    wait_out(obb, sob)

    @pl.when(wid == 0)
    def _():
        pltpu.sync_copy(tail_hbm,
                        out_hbm.at[pl.ds(ROWS128 - TAILROWS, TAILROWS)])


def _sc_body(ids_hbm, table_hbm, out_hbm, idx2, emb3, outb, t0v, gsem):
    wid = lax.axis_index("s") * NC + lax.axis_index("c")

    pltpu.sync_copy(table_hbm.at[pl.ds(0, 1)], t0v)
    t0 = t0v[0]

    lane = lax.iota(jnp.int32, 16)
    hi_mask = lane >= 8

    def chunk_body(c, carry):
        base = wid * ROWS_PER_WORKER + c * CHUNK

        pltpu.sync_copy(ids_hbm.at[pl.ds(base, CHUNK)], idx2)
        copies = [pltpu.async_copy(table_hbm.at[idx2.at[r]],
                                   emb3.at[r], gsem)
                  for r in range(CHUNK)]
        for cp in copies:
            cp.wait()

        for r in range(CHUNK):
            pz = jnp.zeros((16,), jnp.int32)
            for j in range(12):
                v = idx2[r, pl.ds(j * 16, 16)]
                pz = pz + (v == PAD_IDX).astype(jnp.int32)
            v = idx2[r, pl.ds(HIST - 16, 16)]
            pz = pz + ((v == PAD_IDX) & hi_mask).astype(jnp.int32)
            npad = jnp.sum(pz)

            def acc_body(j, accs):
                base8 = j * 8
                return tuple(a + emb3[r, base8 + k]
                             for k, a in enumerate(accs))
            accs = lax.fori_loop(
                0, HIST // 8, acc_body,
                tuple(jnp.zeros((DIM,), jnp.float32) for _ in range(8)))
            a0 = (accs[0] + accs[1]) + (accs[2] + accs[3])
            a1 = (accs[4] + accs[5]) + (accs[6] + accs[7])
            acc = a0 + a1

            cntv = jnp.broadcast_to((HIST - npad).astype(jnp.float32), (DIM,))
            invv = jnp.float32(1.0) / jnp.maximum(cntv, jnp.float32(1.0))
            res = (acc - npad.astype(jnp.float32) * t0) * invv
            outb[r] = jnp.where(cntv > 0, res, jnp.float32(0.0))

        pltpu.sync_copy(outb, out_hbm.at[pl.ds(base, CHUNK)])
        return carry

    lax.fori_loop(0, NCHUNK, chunk_body, 0)


@jax.jit
def _visit_encode(ids, table):
    mesh = plsc.VectorSubcoreMesh(core_axis_name="c", subcore_axis_name="s")
    relayout = pl.kernel(
        _transpose_body,
        out_type=jax.ShapeDtypeStruct((ROWS128, 128), jnp.float32),
        mesh=mesh,
        scratch_types=[
            pltpu.VMEM((DIM, TBLK + 1), jnp.float32),
            pltpu.VMEM((DIM, TBLK + 1), jnp.float32),
            pltpu.VMEM((OROWS, 128), jnp.float32),
            pltpu.VMEM((OROWS, 128), jnp.float32),
            pltpu.SemaphoreType.DMA,
            pltpu.SemaphoreType.DMA,
            pltpu.SemaphoreType.DMA,
            pltpu.SemaphoreType.DMA,
        ],
        compiler_params=pltpu.CompilerParams(needs_layout_passes=False),
    )
    tail = table[MAIN_CODES:].reshape(TAILROWS, 128)
    table_lin = relayout(table.T, tail).reshape(NUM_CODES, DIM)
    ker = pl.kernel(
        _sc_body,
        out_type=jax.ShapeDtypeStruct((BATCH, DIM), jnp.float32),
        mesh=mesh,
        scratch_types=[
            pltpu.VMEM((CHUNK, HIST), jnp.int32),
            pltpu.VMEM((CHUNK, HIST, DIM), jnp.float32),
            pltpu.VMEM((CHUNK, DIM), jnp.float32),
            pltpu.VMEM((1, DIM), jnp.float32),
            pltpu.SemaphoreType.DMA,
        ],
        compiler_params=pltpu.CompilerParams(use_tc_tiling_on_sc=False,
                                             needs_layout_passes=False),
    )
    return ker(ids, table_lin)


def kernel(code_ids_batch, table):
    ids = code_ids_batch.astype(jnp.int32)
    return _visit_encode(ids, table)


# speedup vs baseline: 1.1451x; 1.1451x over previous
import jax
import jax.numpy as jnp
from jax import lax
from jax.experimental import pallas as pl
from jax.experimental.pallas import tpu as pltpu
from jax.experimental.pallas import tpu_sc as plsc

NUM_CODES = 1000000
DIM = 16
PAD_IDX = 0
BATCH = 16384
HIST = 200

NC = 2
NS = 16
NW = NC * NS
ROWS_PER_WORKER = BATCH // NW
CHUNK = 16
NCHUNK = ROWS_PER_WORKER // CHUNK

TBLK = 1024
TSTEPS = 30
NBLK_FULL = TSTEPS * NW
MAIN_CODES = NBLK_FULL * TBLK
TAIL = NUM_CODES - MAIN_CODES
OROWS = TBLK * DIM // 128
ROWS128 = NUM_CODES * DIM // 128
TAILROWS = TAIL * DIM // 128


def _transpose_body(tT_hbm, tail_hbm, out_hbm,
                    tba, tbb, oba, obb, sia, sib, soa, sob):
    wid = lax.axis_index("s") * NC + lax.axis_index("c")
    lanes = lax.iota(jnp.int32, 16)

    def col0(s):
        return (s * NW + wid) * TBLK

    def orow0(s):
        return (s * NW + wid) * OROWS

    def fire_in(s, tb, si):
        pltpu.async_copy(tT_hbm.at[:, pl.ds(col0(s), TBLK)],
                         tb.at[:, pl.ds(0, TBLK)], si)

    def wait_in(tb, si):
        pltpu.make_async_copy(tT_hbm.at[:, pl.ds(0, TBLK)],
                              tb.at[:, pl.ds(0, TBLK)], si).wait()

    def fire_out(s, ob, so):
        pltpu.async_copy(ob, out_hbm.at[pl.ds(orow0(s), OROWS)], so)

    def wait_out(ob, so):
        pltpu.make_async_copy(ob, out_hbm.at[pl.ds(0, OROWS)], so).wait()

    def do_block(tb, ob):
        def kb(j, colv):
            cv = colv
            for kk in range(8):
                e = plsc.load_gather(tb, [lanes, cv])
                ob[j, pl.ds(kk * DIM, DIM)] = e
                cv = cv + 1
            return cv
        lax.fori_loop(0, OROWS, kb, jnp.zeros((16,), jnp.int32))

    fire_in(0, tba, sia)

    def body(i, carry):
        s0 = 2 * i
        s1 = 2 * i + 1
        fire_in(s1, tbb, sib)
        wait_in(tba, sia)

        @pl.when(i > 0)
        def _():
            wait_out(oba, soa)

        do_block(tba, oba)
        fire_out(s0, oba, soa)

        @pl.when(s0 + 2 <= TSTEPS - 1)
        def _():
            fire_in(s0 + 2, tba, sia)

        wait_in(tbb, sib)

        @pl.when(i > 0)
        def _():
            wait_out(obb, sob)

        do_block(tbb, obb)
        fire_out(s1, obb, sob)
        return carry

    lax.fori_loop(0, TSTEPS // 2, body, 0)

    wait_out(oba, soa)
    wait_out(obb, sob)

    @pl.when(wid == 0)
    def _():
        pltpu.sync_copy(tail_hbm,
                        out_hbm.at[pl.ds(ROWS128 - TAILROWS, TAILROWS)])


def _sc_body(ids_hbm, table_hbm, out_hbm,
             idxa, idxb, emba, embb, outa, outd, t0v,
             sia, sib, sga, sgb, soa, sob):
    wid = lax.axis_index("s") * NC + lax.axis_index("c")

    pltpu.sync_copy(table_hbm.at[pl.ds(0, 1)], t0v)
    t0 = t0v[0]

    lane = lax.iota(jnp.int32, 16)
    hi_mask = lane >= 8

    def base_of(s):
        return wid * ROWS_PER_WORKER + s * CHUNK

    def fire_idx(s, idx, si):
        pltpu.async_copy(ids_hbm.at[pl.ds(base_of(s), CHUNK)], idx, si)

    def wait_idx(idx, si):
        pltpu.make_async_copy(ids_hbm.at[pl.ds(0, CHUNK)], idx, si).wait()

    def fire_gathers(idx, emb, sg):
        for r in range(CHUNK):
            pltpu.async_copy(table_hbm.at[idx.at[r]], emb.at[r], sg)

    def wait_gathers(idx, emb, sg):
        for r in range(CHUNK):
            pltpu.make_async_copy(table_hbm.at[idx.at[r]],
                                  emb.at[r], sg).wait()

    def fire_out(s, outb, so):
        pltpu.async_copy(outb, out_hbm.at[pl.ds(base_of(s), CHUNK)], so)

    def wait_out(outb, so):
        pltpu.make_async_copy(outb, out_hbm.at[pl.ds(0, CHUNK)], so).wait()

    def compute(idx2, emb3, outb):
        for r in range(CHUNK):
            pz = jnp.zeros((16,), jnp.int32)
            for j in range(12):
                v = idx2[r, pl.ds(j * 16, 16)]
                pz = pz + (v == PAD_IDX).astype(jnp.int32)
            v = idx2[r, pl.ds(HIST - 16, 16)]
            pz = pz + ((v == PAD_IDX) & hi_mask).astype(jnp.int32)
            npad = jnp.sum(pz)

            def acc_body(j, accs):
                base8 = j * 8
                return tuple(a + emb3[r, base8 + k]
                             for k, a in enumerate(accs))
            accs = lax.fori_loop(
                0, HIST // 8, acc_body,
                tuple(jnp.zeros((DIM,), jnp.float32) for _ in range(8)))
            a0 = (accs[0] + accs[1]) + (accs[2] + accs[3])
            a1 = (accs[4] + accs[5]) + (accs[6] + accs[7])
            acc = a0 + a1

            cntv = jnp.broadcast_to((HIST - npad).astype(jnp.float32), (DIM,))
            invv = jnp.float32(1.0) / jnp.maximum(cntv, jnp.float32(1.0))
            res = (acc - npad.astype(jnp.float32) * t0) * invv
            outb[r] = jnp.where(cntv > 0, res, jnp.float32(0.0))

    fire_idx(0, idxa, sia)
    wait_idx(idxa, sia)
    fire_gathers(idxa, emba, sga)
    fire_idx(1, idxb, sib)

    def body(i, carry):
        s0 = 2 * i
        s1 = 2 * i + 1

        wait_gathers(idxa, emba, sga)
        wait_idx(idxb, sib)
        fire_gathers(idxb, embb, sgb)

        @pl.when(i > 0)
        def _():
            wait_out(outa, soa)

        compute(idxa, emba, outa)

        @pl.when(s0 + 2 <= NCHUNK - 1)
        def _():
            fire_idx(s0 + 2, idxa, sia)

        fire_out(s0, outa, soa)

        wait_gathers(idxb, embb, sgb)

        @pl.when(s1 + 1 <= NCHUNK - 1)
        def _():
            wait_idx(idxa, sia)
            fire_gathers(idxa, emba, sga)

        @pl.when(i > 0)
        def _():
            wait_out(outd, sob)

        compute(idxb, embb, outd)

        @pl.when(s1 + 2 <= NCHUNK - 1)
        def _():
            fire_idx(s1 + 2, idxb, sib)

        fire_out(s1, outd, sob)
        return carry

    lax.fori_loop(0, NCHUNK // 2, body, 0)
    wait_out(outa, soa)
    wait_out(outd, sob)


@jax.jit
def _visit_encode(ids, table):
    mesh = plsc.VectorSubcoreMesh(core_axis_name="c", subcore_axis_name="s")
    relayout = pl.kernel(
        _transpose_body,
        out_type=jax.ShapeDtypeStruct((ROWS128, 128), jnp.float32),
        mesh=mesh,
        scratch_types=[
            pltpu.VMEM((DIM, TBLK + 1), jnp.float32),
            pltpu.VMEM((DIM, TBLK + 1), jnp.float32),
            pltpu.VMEM((OROWS, 128), jnp.float32),
            pltpu.VMEM((OROWS, 128), jnp.float32),
            pltpu.SemaphoreType.DMA,
            pltpu.SemaphoreType.DMA,
            pltpu.SemaphoreType.DMA,
            pltpu.SemaphoreType.DMA,
        ],
        compiler_params=pltpu.CompilerParams(needs_layout_passes=False),
    )
    tail = table[MAIN_CODES:].reshape(TAILROWS, 128)
    table_lin = relayout(table.T, tail).reshape(NUM_CODES, DIM)
    ker = pl.kernel(
        _sc_body,
        out_type=jax.ShapeDtypeStruct((BATCH, DIM), jnp.float32),
        mesh=mesh,
        scratch_types=[
            pltpu.VMEM((CHUNK, HIST), jnp.int32),
            pltpu.VMEM((CHUNK, HIST), jnp.int32),
            pltpu.VMEM((CHUNK, HIST, DIM), jnp.float32),
            pltpu.VMEM((CHUNK, HIST, DIM), jnp.float32),
            pltpu.VMEM((CHUNK, DIM), jnp.float32),
            pltpu.VMEM((CHUNK, DIM), jnp.float32),
            pltpu.VMEM((1, DIM), jnp.float32),
            pltpu.SemaphoreType.DMA,
            pltpu.SemaphoreType.DMA,
            pltpu.SemaphoreType.DMA,
            pltpu.SemaphoreType.DMA,
            pltpu.SemaphoreType.DMA,
            pltpu.SemaphoreType.DMA,
        ],
        compiler_params=pltpu.CompilerParams(use_tc_tiling_on_sc=False,
                                             needs_layout_passes=False),
    )
    return ker(ids, table_lin)


def kernel(code_ids_batch, table):
    ids = code_ids_batch.astype(jnp.int32)
    return _visit_encode(ids, table)
